# baseline (speedup 1.0000x reference)
import functools
import math

import jax
import jax.numpy as jnp
from jax import lax
from jax.experimental import pallas as pl
from jax.experimental.pallas import tpu as pltpu
from jax.experimental.pallas import tpu_sc as plsc

_B, _L, _OMEGA = 4096, 200, 64
_VOCAB = 256
_MAX_SEQ_LEN = 512
_D = 2 * _OMEGA
_T = _B * _L


def _prep_body(tab_ref, w_ref, b_ref, g2_ref, cs_ref):
    t = tab_ref[...]
    z = lax.dot_general(t, w_ref[...], (((1,), (1,)), ((), ())),
                        preferred_element_type=jnp.float32)
    g2_ref[...] = t * jax.nn.sigmoid(z + b_ref[...])
    alpha = 2.0 * math.pi / _MAX_SEQ_LEN
    row = lax.broadcasted_iota(jnp.int32, (8, _VOCAB), 0)
    col = lax.broadcasted_iota(jnp.int32, (8, _VOCAB), 1)
    phi = alpha * (col % _L).astype(jnp.float32)
    cs_ref[...] = jnp.where(row == 0, jnp.cos(phi), jnp.sin(phi))


def _prep(table, W_gate, b_gate):
    return pl.pallas_call(
        _prep_body,
        grid=(1,),
        in_specs=[
            pl.BlockSpec((_VOCAB, _OMEGA), lambda i: (0, 0)),
            pl.BlockSpec((_OMEGA, _OMEGA), lambda i: (0, 0)),
            pl.BlockSpec((1, _OMEGA), lambda i: (0, 0)),
        ],
        out_specs=[
            pl.BlockSpec((_VOCAB, _OMEGA), lambda i: (0, 0)),
            pl.BlockSpec((8, _VOCAB), lambda i: (0, 0)),
        ],
        out_shape=[
            jax.ShapeDtypeStruct((_VOCAB, _OMEGA), jnp.float32),
            jax.ShapeDtypeStruct((8, _VOCAB), jnp.float32),
        ],
    )(table, W_gate, b_gate.reshape(1, _OMEGA))


_NC, _NS = 2, 16
_NW = _NC * _NS
_TPW = _T // _NW
_H = 128
_NU = _TPW // _H
_HW = _H * _D
_G2W = _VOCAB * _OMEGA


@functools.cache
def _build_lookup():
    mesh = plsc.VectorSubcoreMesh(core_axis_name="c", subcore_axis_name="s")
    return functools.partial(
        pl.kernel,
        mesh=mesh,
        compiler_params=pltpu.CompilerParams(needs_layout_passes=False),
        out_type=jax.ShapeDtypeStruct((_T * _D,), jnp.float32),
        scratch_types=[
            pltpu.VMEM((_TPW,), jnp.int32),
            pltpu.VMEM((_G2W,), jnp.float32),
            pltpu.VMEM((2 * _VOCAB,), jnp.float32),
            pltpu.VMEM((_HW,), jnp.float32),
            pltpu.VMEM((_HW,), jnp.float32),
            pltpu.SemaphoreType.DMA,
            pltpu.SemaphoreType.DMA,
        ],
    )(_lookup_body)


def _lookup_body(x_hbm, g2_hbm, cs_hbm, out_hbm, xb, g2f, csf, st0, st1,
                 ss0, ss1):
    st = (st0, st1)
    ss = (ss0, ss1)
    wid = lax.axis_index("s") * _NC + lax.axis_index("c")
    base = wid * _TPW

    pltpu.sync_copy(x_hbm.at[pl.ds(base, _TPW)], xb)
    pltpu.sync_copy(g2_hbm, g2f)
    pltpu.sync_copy(cs_hbm.at[pl.ds(0, 2 * _VOCAB)], csf)

    def build(u, s):
        def grp_body(grp, carry):
            o = u * _H + grp * 16
            iot = lax.iota(jnp.int32, 16)
            vvec = xb[pl.ds(o, 16)]
            lvec = (base + o + iot) % _L
            cvec = plsc.load_gather(csf, [lvec])
            svec = plsc.load_gather(csf, [lvec + _VOCAB])
            vbase = vvec * _OMEGA
            dstb = grp * (16 * _D) + iot * _D
            for c in range(_OMEGA):
                col = plsc.load_gather(g2f, [vbase + c])
                plsc.store_scatter(st[s], [dstb + c], col * cvec)
                plsc.store_scatter(st[s], [dstb + (_OMEGA + c)], col * svec)
            return carry

        lax.fori_loop(0, _H // 16, grp_body, 0)

    def fire_scatter(u, s):
        pltpu.async_copy(st[s], out_hbm.at[pl.ds((base + u * _H) * _D, _HW)],
                         ss[s])

    def wait_scatter(s):
        pltpu.make_async_copy(st[s], out_hbm.at[pl.ds(base * _D, _HW)],
                              ss[s]).wait()

    def body(i2, carry):
        for b in range(2):
            u = i2 * 2 + b

            @pl.when(u >= 2)
            def _():
                wait_scatter(b)

            build(u, b)
            fire_scatter(u, b)
        return carry

    lax.fori_loop(0, _NU // 2, body, 0)
    wait_scatter(0)
    wait_scatter(1)


def kernel(x, table, W_gate, b_gate):
    g2, cs = _prep(table, W_gate, b_gate)
    out = _build_lookup()(x.reshape(_T), g2.reshape(_G2W),
                          cs.reshape(8 * _VOCAB))
    return out.reshape(_B, _L, _D)


# speedup vs baseline: 8.4639x; 8.4639x over previous
import functools
import math

import jax
import jax.numpy as jnp
from jax import lax
from jax.experimental import pallas as pl
from jax.experimental.pallas import tpu as pltpu
from jax.experimental.pallas import tpu_sc as plsc

_B, _L, _OMEGA = 4096, 200, 64
_VOCAB = 256
_MAX_SEQ_LEN = 512
_D = 2 * _OMEGA
_T = _B * _L

_LBLK = 8


def _expand_body(tab_ref, w_ref, b_ref, out_ref):
    t = tab_ref[...]
    z = lax.dot_general(t, w_ref[...], (((1,), (1,)), ((), ())),
                        preferred_element_type=jnp.float32)
    g = t * jax.nn.sigmoid(z + b_ref[...])
    i = pl.program_id(0)
    alpha = 2.0 * math.pi / _MAX_SEQ_LEN
    pos = i * _LBLK + lax.broadcasted_iota(jnp.int32, (_LBLK, 1, 1), 0)
    phi = alpha * pos.astype(jnp.float32)
    out_ref[:, :, 0:_OMEGA] = g[None, :, :] * jnp.cos(phi)
    out_ref[:, :, _OMEGA:_D] = g[None, :, :] * jnp.sin(phi)


def _expand(table, W_gate, b_gate):
    return pl.pallas_call(
        _expand_body,
        grid=(_L // _LBLK,),
        in_specs=[
            pl.BlockSpec((_VOCAB, _OMEGA), lambda i: (0, 0)),
            pl.BlockSpec((_OMEGA, _OMEGA), lambda i: (0, 0)),
            pl.BlockSpec((1, _OMEGA), lambda i: (0, 0)),
        ],
        out_specs=pl.BlockSpec((_LBLK, _VOCAB, _D), lambda i: (i, 0, 0)),
        out_shape=jax.ShapeDtypeStruct((_L, _VOCAB, _D), jnp.float32),
    )(table, W_gate, b_gate.reshape(1, _OMEGA))


_NC, _NS = 2, 16
_NW = _NC * _NS
_TPW = _T // _NW
_H = 128
_NU = _TPW // _H
_NB = 4


@functools.cache
def _build_lookup():
    mesh = plsc.VectorSubcoreMesh(core_axis_name="c", subcore_axis_name="s")
    return functools.partial(
        pl.kernel,
        mesh=mesh,
        out_type=jax.ShapeDtypeStruct((_T, _D), jnp.float32),
        scratch_types=[
            pltpu.VMEM((_TPW,), jnp.int32),
            pltpu.VMEM((_NB, _H, _D), jnp.float32),
            pltpu.SemaphoreType.DMA,
            pltpu.SemaphoreType.DMA,
            pltpu.SemaphoreType.DMA,
            pltpu.SemaphoreType.DMA,
            pltpu.SemaphoreType.DMA,
            pltpu.SemaphoreType.DMA,
            pltpu.SemaphoreType.DMA,
            pltpu.SemaphoreType.DMA,
        ],
    )(_lookup_body)


def _lookup_body(x_hbm, g_hbm, out_hbm, xi, rb,
                 sg0, sg1, sg2, sg3, ss0, ss1, ss2, ss3):
    sg = (sg0, sg1, sg2, sg3)
    ss = (ss0, ss1, ss2, ss3)
    wid = lax.axis_index("s") * _NC + lax.axis_index("c")
    base = wid * _TPW

    pltpu.sync_copy(x_hbm.at[pl.ds(base, _TPW)], xi)

    def idx_body(j, carry):
        o16 = j * 16
        tvec = base + o16 + lax.iota(jnp.int32, 16)
        xi[pl.ds(o16, 16)] = xi[pl.ds(o16, 16)] + (tvec % _L) * _VOCAB
        return carry

    lax.fori_loop(0, _TPW // 16, idx_body, 0)

    def fire_gather(i, s):
        pltpu.async_copy(g_hbm.at[xi.at[pl.ds(i * _H, _H)]], rb.at[s], sg[s])

    def wait_gather(i, s):
        pltpu.make_async_copy(g_hbm.at[xi.at[pl.ds(i * _H, _H)]],
                              rb.at[s], sg[s]).wait()

    def fire_scatter(i, s):
        pltpu.async_copy(rb.at[s], out_hbm.at[pl.ds(base + i * _H, _H)], ss[s])

    def wait_scatter(s):
        pltpu.make_async_copy(rb.at[s], out_hbm.at[pl.ds(base, _H)],
                              ss[s]).wait()

    for i in range(_NB - 1):
        fire_gather(i, i)

    def body(i4, carry):
        for b in range(_NB):
            i = i4 * _NB + b
            wait_gather(i, b)
            fire_scatter(i, b)
            nxt = i + _NB - 1
            s2 = (b + _NB - 1) % _NB

            @pl.when(nxt < _NU)
            def _():
                @pl.when(i >= 1)
                def _():
                    wait_scatter(s2)

                fire_gather(nxt, s2)

        return carry

    lax.fori_loop(0, _NU // _NB, body, 0)
    for s in range(_NB):
        wait_scatter(s)


def kernel(x, table, W_gate, b_gate):
    G = _expand(table, W_gate, b_gate).reshape(_L * _VOCAB, _D)
    out = _build_lookup()(x.reshape(_T), G)
    return out.reshape(_B, _L, _D)


# speedup vs baseline: 8.6479x; 1.0217x over previous
import functools
import math

import jax
import jax.numpy as jnp
from jax import lax
from jax.experimental import pallas as pl
from jax.experimental.pallas import tpu as pltpu
from jax.experimental.pallas import tpu_sc as plsc

_B, _L, _OMEGA = 4096, 200, 64
_VOCAB = 256
_MAX_SEQ_LEN = 512
_D = 2 * _OMEGA
_T = _B * _L

_LBLK = 40


def _expand_body(tab_ref, w_ref, b_ref, out_ref):
    t = tab_ref[...]
    z = lax.dot_general(t, w_ref[...], (((1,), (1,)), ((), ())),
                        preferred_element_type=jnp.float32)
    g = t * jax.nn.sigmoid(z + b_ref[...])
    i = pl.program_id(0)
    alpha = 2.0 * math.pi / _MAX_SEQ_LEN
    pos = i * _LBLK + lax.broadcasted_iota(jnp.int32, (_LBLK, 1, 1), 0)
    phi = alpha * pos.astype(jnp.float32)
    out_ref[:, :, 0:_OMEGA] = g[None, :, :] * jnp.cos(phi)
    out_ref[:, :, _OMEGA:_D] = g[None, :, :] * jnp.sin(phi)


def _expand(table, W_gate, b_gate):
    return pl.pallas_call(
        _expand_body,
        grid=(_L // _LBLK,),
        in_specs=[
            pl.BlockSpec((_VOCAB, _OMEGA), lambda i: (0, 0)),
            pl.BlockSpec((_OMEGA, _OMEGA), lambda i: (0, 0)),
            pl.BlockSpec((1, _OMEGA), lambda i: (0, 0)),
        ],
        out_specs=pl.BlockSpec((_LBLK, _VOCAB, _D), lambda i: (i, 0, 0)),
        out_shape=jax.ShapeDtypeStruct((_L, _VOCAB, _D), jnp.float32),
    )(table, W_gate, b_gate.reshape(1, _OMEGA))


_NC, _NS = 2, 16
_NW = _NC * _NS
_TPW = _T // _NW
_H = 128
_NU = _TPW // _H
_NB = 4


@functools.cache
def _build_lookup():
    mesh = plsc.VectorSubcoreMesh(core_axis_name="c", subcore_axis_name="s")
    return functools.partial(
        pl.kernel,
        mesh=mesh,
        out_type=jax.ShapeDtypeStruct((_T, _D), jnp.float32),
        scratch_types=[
            pltpu.VMEM((_TPW,), jnp.int32),
            pltpu.VMEM((_NB, _H, _D), jnp.float32),
            pltpu.SemaphoreType.DMA,
            pltpu.SemaphoreType.DMA,
            pltpu.SemaphoreType.DMA,
            pltpu.SemaphoreType.DMA,
            pltpu.SemaphoreType.DMA,
            pltpu.SemaphoreType.DMA,
            pltpu.SemaphoreType.DMA,
            pltpu.SemaphoreType.DMA,
        ],
    )(_lookup_body)


def _lookup_body(x_hbm, g_hbm, out_hbm, xi, rb,
                 sg0, sg1, sg2, sg3, ss0, ss1, ss2, ss3):
    sg = (sg0, sg1, sg2, sg3)
    ss = (ss0, ss1, ss2, ss3)
    wid = lax.axis_index("s") * _NC + lax.axis_index("c")
    base = wid * _TPW

    pltpu.sync_copy(x_hbm.at[pl.ds(base, _TPW)], xi)

    def idx_body(j, carry):
        o16 = j * 16
        tvec = base + o16 + lax.iota(jnp.int32, 16)
        xi[pl.ds(o16, 16)] = xi[pl.ds(o16, 16)] + (tvec % _L) * _VOCAB
        return carry

    lax.fori_loop(0, _TPW // 16, idx_body, 0)

    def fire_gather(i, s):
        pltpu.async_copy(g_hbm.at[xi.at[pl.ds(i * _H, _H)]], rb.at[s], sg[s])

    def wait_gather(i, s):
        pltpu.make_async_copy(g_hbm.at[xi.at[pl.ds(i * _H, _H)]],
                              rb.at[s], sg[s]).wait()

    def fire_scatter(i, s):
        pltpu.async_copy(rb.at[s], out_hbm.at[pl.ds(base + i * _H, _H)], ss[s])

    def wait_scatter(s):
        pltpu.make_async_copy(rb.at[s], out_hbm.at[pl.ds(base, _H)],
                              ss[s]).wait()

    for i in range(_NB - 1):
        fire_gather(i, i)

    def body(i4, carry):
        for b in range(_NB):
            i = i4 * _NB + b
            wait_gather(i, b)
            fire_scatter(i, b)
            nxt = i + _NB - 1
            s2 = (b + _NB - 1) % _NB

            @pl.when(nxt < _NU)
            def _():
                @pl.when(i >= 1)
                def _():
                    wait_scatter(s2)

                fire_gather(nxt, s2)

        return carry

    lax.fori_loop(0, _NU // _NB, body, 0)
    for s in range(_NB):
        wait_scatter(s)


def kernel(x, table, W_gate, b_gate):
    G = _expand(table, W_gate, b_gate).reshape(_L * _VOCAB, _D)
    out = _build_lookup()(x.reshape(_T), G)
    return out.reshape(_B, _L, _D)
